# baseline (speedup 1.0000x reference)
import jax
import jax.numpy as jnp
from jax import lax
from jax.experimental import pallas as pl
from jax.experimental.pallas import tpu as pltpu
from jax.experimental.pallas import tpu_sc as plsc

H = 256
HH = 128


def _mm_kernel(x_ref, w_ref, b_ref, o_ref, o2_ref):
    res = (
        jnp.dot(x_ref[...], w_ref[...], preferred_element_type=jnp.float32)
        + b_ref[...]
    )
    o_ref[...] = res
    o2_ref[0] = res[:, :HH]
    o2_ref[1] = res[:, HH:2 * HH]


def _matmul_bias(x, w, b, block_m=400):
    m, k = x.shape
    n = w.shape[1]
    assert m % block_m == 0
    return pl.pallas_call(
        _mm_kernel,
        grid=(m // block_m,),
        in_specs=[
            pl.BlockSpec((block_m, k), lambda i: (i, 0)),
            pl.BlockSpec((k, n), lambda i: (0, 0)),
            pl.BlockSpec((1, n), lambda i: (0, 0)),
        ],
        out_specs=[
            pl.BlockSpec((block_m, n), lambda i: (i, 0)),
            pl.BlockSpec((2, block_m, HH), lambda i: (0, i, 0)),
        ],
        out_shape=[
            jax.ShapeDtypeStruct((m, n), jnp.float32),
            jax.ShapeDtypeStruct((2, m, HH), jnp.float32),
        ],
    )(x, w, b.reshape(1, n))


def _pz_kernel(s_ref, t_ref, b_ref, dg_ref, w_ref, bias_ref, o_ref):
    z = (jnp.concatenate([s_ref[0], s_ref[1]], axis=-1)
         + t_ref[...] - b_ref[...] * dg_ref[...])
    o_ref[0] = (
        jnp.dot(z, w_ref[...], preferred_element_type=jnp.float32)
        + bias_ref[0]
    )


def _p_matmul(s_pair, pcd, t_col, b_col, degw, w, bias, block_m=400):
    n = s_pair.shape[1]
    assert n % block_m == 0
    return pl.pallas_call(
        _pz_kernel,
        grid=(n // block_m, 2),
        in_specs=[
            pl.BlockSpec((2, block_m, HH), lambda i, j: (0, i, 0)),
            pl.BlockSpec((block_m, H), lambda i, j, c=t_col: (i, c)),
            pl.BlockSpec((block_m, H), lambda i, j, c=b_col: (i, c)),
            pl.BlockSpec((block_m, 1), lambda i, j: (i, 0)),
            pl.BlockSpec((H, HH), lambda i, j: (0, j)),
            pl.BlockSpec((1, 1, HH), lambda i, j: (j, 0, 0)),
        ],
        out_specs=pl.BlockSpec((1, block_m, HH), lambda i, j: (j, i, 0)),
        out_shape=jax.ShapeDtypeStruct((2, n, HH), jnp.float32),
    )(s_pair, pcd, pcd, degw.reshape(n, 1), w, bias.reshape(2, 1, HH))


def _fin_kernel(g_ref, w2_ref, b2_ref, o_ref):
    q = jax.nn.sigmoid(jnp.concatenate([g_ref[0], g_ref[1]], axis=-1))
    o_ref[...] = jax.nn.sigmoid(
        jnp.sum(q * w2_ref[...], axis=1, keepdims=True) + b2_ref[...])


def _final_stage(g_pair, w2, b2, block_m=2000):
    m = g_pair.shape[1]
    assert m % block_m == 0
    out = pl.pallas_call(
        _fin_kernel,
        grid=(m // block_m,),
        in_specs=[
            pl.BlockSpec((2, block_m, HH), lambda i: (0, i, 0)),
            pl.BlockSpec((1, H), lambda i: (0, 0)),
            pl.BlockSpec((1, 1), lambda i: (0, 0)),
        ],
        out_specs=pl.BlockSpec((block_m, 1), lambda i: (i, 0)),
        out_shape=jax.ShapeDtypeStruct((m, 1), jnp.float32),
    )(g_pair, w2.reshape(1, H), b2.reshape(1, 1))
    return out.reshape(-1)


_EC = 128


def _edge_phase(a_hbm, src_hbm, dst_hbm, w_hbm, s_out, degw_out,
                src_v, dst_v, w_v, rows_v, src_t, dst_t, w_t, rows_t,
                zdeg_v, acc, degw_acc, dvs_v, wvs_v,
                sem_i0, sem_i1, sem_g0, sem_g1, sem_s0, sem_s1,
                sem_d0, sem_d1):
    c = lax.axis_index("c")
    s = lax.axis_index("s")
    n = a_hbm.shape[1]
    e = src_hbm.shape[0]
    per_sub = e // 16
    nfull = per_sub // _EC
    tail = per_sub - nfull * _EC
    rows_a = (n // 16) // 8 * 8
    rows_rem = n - 16 * rows_a
    nzc = rows_a // _EC
    zrem = rows_a - nzc * _EC

    zero16 = jnp.zeros((16,), jnp.float32)

    def _zrow2(i, _):
        rows_v[0, i // 8, pl.ds((i % 8) * 16, 16)] = zero16
        rows_v[1, i // 8, pl.ds((i % 8) * 16, 16)] = zero16
        return 0
    lax.fori_loop(0, _EC * 8, _zrow2, 0)

    def _zdeg(i, _):
        zdeg_v[pl.ds(i * 16, 16)] = zero16
        return 0
    lax.fori_loop(0, zdeg_v.shape[0] // 16, _zdeg, 0)

    for q in range(nzc):
        pltpu.sync_copy(rows_v.at[0],
                        acc.at[pl.ds(s * rows_a + q * _EC, _EC)])
    pltpu.sync_copy(rows_v.at[0].at[pl.ds(0, zrem)],
                    acc.at[pl.ds(s * rows_a + nzc * _EC, zrem)])

    @pl.when(s == 15)
    def _():
        pltpu.sync_copy(rows_v.at[0].at[pl.ds(0, rows_rem)],
                        acc.at[pl.ds(16 * rows_a, rows_rem)])

    @pl.when(jnp.logical_and(c == 0, s < 10))
    def _():
        pltpu.sync_copy(zdeg_v.at[pl.ds(0, 1000)],
                        degw_acc.at[pl.ds(s * 1000, 1000)])

    plsc.subcore_barrier()

    def _scale(rows, w_ref, k):
        def body(r16, _):
            wrow = w_ref[pl.ds(r16 * 16, 16)]
            for t in range(16):
                r = r16 * 16 + t
                wv = lax.gather(
                    wrow, jnp.full((16, 1), t, jnp.int32),
                    lax.GatherDimensionNumbers(
                        offset_dims=(), collapsed_slice_dims=(0,),
                        start_index_map=(0,)),
                    (1,), mode=lax.GatherScatterMode.PROMISE_IN_BOUNDS)
                for j in range(8):
                    sl = pl.ds(j * 16, 16)
                    rows[r, sl] = rows[r, sl] * wv
            return 0
        lax.fori_loop(0, k // 16, body, 0)

    sv = (src_v.at[0], src_v.at[1])
    dv = (dst_v.at[0], dst_v.at[1])
    wv = (w_v.at[0], w_v.at[1])
    rv = (rows_v.at[0], rows_v.at[1])
    dvs = (dvs_v.at[0], dvs_v.at[1])
    wvs = (wvs_v.at[0], wvs_v.at[1])
    semi = (sem_i0, sem_i1)
    semg = (sem_g0, sem_g1)
    sems = (sem_s0, sem_s1)
    semd = (sem_d0, sem_d1)

    def base_of(g):
        return s * per_sub + jnp.minimum(g, nfull - 1) * _EC

    def fetch_idx(g, p):
        b = base_of(g)
        pltpu.async_copy(src_hbm.at[pl.ds(b, _EC)], sv[p], semi[p])
        pltpu.async_copy(dst_hbm.at[pl.ds(b, _EC)], dv[p], semi[p])
        pltpu.async_copy(w_hbm.at[pl.ds(b, _EC)], wv[p], semi[p])

    def wait_idx(p):
        b = base_of(0)
        pltpu.make_async_copy(src_hbm.at[pl.ds(b, _EC)], sv[p], semi[p]).wait()
        pltpu.make_async_copy(dst_hbm.at[pl.ds(b, _EC)], dv[p], semi[p]).wait()
        pltpu.make_async_copy(w_hbm.at[pl.ds(b, _EC)], wv[p], semi[p]).wait()

    def gather_rows(p):
        pltpu.async_copy(a_hbm.at[c].at[sv[p]], rv[p], semg[p])

    def wait_rows(p):
        pltpu.make_async_copy(a_hbm.at[c].at[sv[p]], rv[p], semg[p]).wait()

    def snap_idx(p):
        for j in range(8):
            sl = pl.ds(j * 16, 16)
            dvs[p][sl] = dv[p][sl]

        @pl.when(c == 0)
        def _():
            for j in range(8):
                sl = pl.ds(j * 16, 16)
                wvs[p][sl] = wv[p][sl]

    def scatter(p):
        pltpu.async_copy(rv[p], acc.at[dvs[p]], sems[p], add=True)

        @pl.when(c == 0)
        def _():
            pltpu.async_copy(wvs[p], degw_acc.at[dvs[p]], semd[p], add=True)

    def wait_scatter(p):
        pltpu.make_async_copy(rv[p], acc.at[dvs[p]], sems[p]).wait()

        @pl.when(c == 0)
        def _():
            pltpu.make_async_copy(wvs[p], degw_acc.at[dvs[p]],
                                  semd[p]).wait()

    def half_step(i, p):
        wait_rows(p)
        _scale(rv[p], wv[p], _EC)
        snap_idx(p)
        wait_idx(1 - p)

        @pl.when(i >= 1)
        def _():
            wait_scatter(1 - p)
        gather_rows(1 - p)
        scatter(p)
        fetch_idx(i + 2, p)

    assert nfull % 2 == 0
    fetch_idx(0, 0)
    wait_idx(0)
    gather_rows(0)
    fetch_idx(1, 1)

    def _main(u, _):
        half_step(2 * u, 0)
        half_step(2 * u + 1, 1)
        return 0
    lax.fori_loop(0, nfull // 2, _main, 0)
    wait_rows(0)
    wait_idx(1)
    wait_scatter(1)

    if tail:
        b = s * per_sub + nfull * _EC
        pltpu.sync_copy(src_hbm.at[pl.ds(b, tail)], src_t)
        pltpu.sync_copy(dst_hbm.at[pl.ds(b, tail)], dst_t)
        pltpu.sync_copy(w_hbm.at[pl.ds(b, tail)], w_t)
        pltpu.async_copy(a_hbm.at[c].at[src_t], rows_t, sem_g0).wait()
        _scale(rows_t, w_t, tail)
        pltpu.sync_copy(rows_t, acc.at[dst_t], add=True)

        @pl.when(c == 0)
        def _():
            pltpu.sync_copy(w_t, degw_acc.at[dst_t], add=True)

    plsc.subcore_barrier()

    for q in range(nzc):
        lo = s * rows_a + q * _EC
        pltpu.sync_copy(acc.at[pl.ds(lo, _EC)], rows_v.at[0])
        pltpu.sync_copy(rows_v.at[0], s_out.at[c].at[pl.ds(lo, _EC)])
    lo2 = s * rows_a + nzc * _EC
    pltpu.sync_copy(acc.at[pl.ds(lo2, zrem)], rows_v.at[0].at[pl.ds(0, zrem)])
    pltpu.sync_copy(rows_v.at[0].at[pl.ds(0, zrem)],
                    s_out.at[c].at[pl.ds(lo2, zrem)])

    @pl.when(s == 15)
    def _():
        pltpu.sync_copy(acc.at[pl.ds(16 * rows_a, rows_rem)],
                        rows_v.at[1].at[pl.ds(0, rows_rem)])
        pltpu.sync_copy(rows_v.at[1].at[pl.ds(0, rows_rem)],
                        s_out.at[c].at[pl.ds(16 * rows_a, rows_rem)])

    @pl.when(jnp.logical_and(c == 0, s < 10))
    def _():
        pltpu.sync_copy(degw_acc.at[pl.ds(s * 1000, 1000)],
                        zdeg_v.at[pl.ds(0, 1000)])
        pltpu.sync_copy(zdeg_v.at[pl.ds(0, 1000)],
                        degw_out.at[pl.ds(s * 1000, 1000)])


def _edge2_body(a1, src1, dst1, w1, a2, src2, dst2, w2,
                s1_out, dg1_out, s2_out, dg2_out, *scr):
    _edge_phase(a1, src1, dst1, w1, s1_out, dg1_out, *scr)
    _edge_phase(a2, src2, dst2, w2, s2_out, dg2_out, *scr)


def _edge_stage(a1_pair, src1, dst1, w1, a2_pair, src2, dst2, w2):
    a_pair, src = a1_pair, src1
    n = a_pair.shape[1]
    e = src.shape[0]
    assert a2_pair.shape == a1_pair.shape and src2.shape == src1.shape
    assert e % 16 == 0 and (e // 16) % 8 == 0
    assert n % 16 == 0 and n % 1000 == 0
    tail_n = max(e // 16 % _EC, 8)
    mesh = plsc.VectorSubcoreMesh(core_axis_name="c", subcore_axis_name="s")
    fn = pl.kernel(
        _edge2_body,
        out_type=[
            jax.ShapeDtypeStruct((2, n, HH), jnp.float32),
            jax.ShapeDtypeStruct((n,), jnp.float32),
            jax.ShapeDtypeStruct((2, n, HH), jnp.float32),
            jax.ShapeDtypeStruct((n,), jnp.float32),
        ],
        mesh=mesh,
        scratch_types=[
            pltpu.VMEM((2, _EC), jnp.int32),
            pltpu.VMEM((2, _EC), jnp.int32),
            pltpu.VMEM((2, _EC), jnp.float32),
            pltpu.VMEM((2, _EC, HH), jnp.float32),
            pltpu.VMEM((tail_n,), jnp.int32),
            pltpu.VMEM((tail_n,), jnp.int32),
            pltpu.VMEM((tail_n,), jnp.float32),
            pltpu.VMEM((tail_n, HH), jnp.float32),
            pltpu.VMEM((1040,), jnp.float32),
            pltpu.VMEM_SHARED((n, HH), jnp.float32),
            pltpu.VMEM_SHARED((n,), jnp.float32),
            pltpu.VMEM((2, _EC), jnp.int32),
            pltpu.VMEM((2, _EC), jnp.float32),
            pltpu.SemaphoreType.DMA,
            pltpu.SemaphoreType.DMA,
            pltpu.SemaphoreType.DMA,
            pltpu.SemaphoreType.DMA,
            pltpu.SemaphoreType.DMA,
            pltpu.SemaphoreType.DMA,
            pltpu.SemaphoreType.DMA,
            pltpu.SemaphoreType.DMA,
        ],
    )
    return fn(a1_pair, src1, dst1, w1, a2_pair, src2, dst2, w2)


def _dec_body(p1_hbm, p2_hbm, r1_hbm, r2_hbm, g_out,
              i1_v, i2_v, g1_v, g2_v, i1_t, i2_t, g1_t, g2_t,
              sem_i0, sem_i1, sem_g0, sem_g1, sem_o0, sem_o1):
    c = lax.axis_index("c")
    s = lax.axis_index("s")
    l = r1_hbm.shape[0]
    nfull = l // _EC
    tail = l - nfull * _EC
    n_t = (nfull - s + 15) // 16

    i1 = (i1_v.at[0], i1_v.at[1])
    i2 = (i2_v.at[0], i2_v.at[1])
    g1 = (g1_v.at[0], g1_v.at[1])
    g2 = (g2_v.at[0], g2_v.at[1])
    semi = (sem_i0, sem_i1)
    semg = (sem_g0, sem_g1)
    semo = (sem_o0, sem_o1)

    def base_of(t):
        return (s + 16 * jnp.minimum(t, n_t - 1)) * _EC

    def _add(ga, gb, k):
        def body(r, _):
            for j in range(8):
                sl = pl.ds(j * 16, 16)
                ga[r, sl] = ga[r, sl] + gb[r, sl]
            return 0
        lax.fori_loop(0, k, body, 0)

    def fetch_idx(t, p):
        b = base_of(t)
        pltpu.async_copy(r1_hbm.at[pl.ds(b, _EC)], i1[p], semi[p])
        pltpu.async_copy(r2_hbm.at[pl.ds(b, _EC)], i2[p], semi[p])

    def wait_idx(p):
        pltpu.make_async_copy(r1_hbm.at[pl.ds(0, _EC)], i1[p], semi[p]).wait()
        pltpu.make_async_copy(r2_hbm.at[pl.ds(0, _EC)], i2[p], semi[p]).wait()

    def gathers(p):
        pltpu.async_copy(p1_hbm.at[c].at[i1[p]], g1[p], semg[p])
        pltpu.async_copy(p2_hbm.at[c].at[i2[p]], g2[p], semg[p])

    def wait_gathers(p):
        pltpu.make_async_copy(p1_hbm.at[c].at[i1[p]], g1[p], semg[p]).wait()
        pltpu.make_async_copy(p2_hbm.at[c].at[i2[p]], g2[p], semg[p]).wait()

    def wait_store(p):
        pltpu.make_async_copy(g_out.at[c].at[pl.ds(0, _EC)],
                              g1[p], semo[p]).wait()

    def step(t, p):
        wait_gathers(p)
        _add(g1[p], g2[p], _EC)
        wait_idx(1 - p)

        @pl.when(t >= 1)
        def _():
            wait_store(1 - p)
        gathers(1 - p)
        pltpu.async_copy(g1[p], g_out.at[c].at[pl.ds(base_of(t), _EC)],
                         semo[p])
        fetch_idx(t + 2, p)

    fetch_idx(0, 0)
    wait_idx(0)
    gathers(0)
    fetch_idx(1, 1)

    nslots = (((nfull + 15) // 16) + 1) // 2 * 2

    def _main(u, _):
        step(2 * u, 0)
        step(2 * u + 1, 1)
        return 0
    lax.fori_loop(0, nslots // 2, _main, 0)
    wait_gathers(0)
    wait_idx(1)
    wait_store(1)

    if tail:
        @pl.when(s == 15)
        def _():
            b = nfull * _EC
            pltpu.sync_copy(r1_hbm.at[pl.ds(b, tail)], i1_t)
            pltpu.sync_copy(r2_hbm.at[pl.ds(b, tail)], i2_t)
            pltpu.async_copy(p1_hbm.at[c].at[i1_t], g1_t, sem_g0).wait()
            pltpu.async_copy(p2_hbm.at[c].at[i2_t], g2_t, sem_g0).wait()
            _add(g1_t, g2_t, tail)
            pltpu.sync_copy(g1_t, g_out.at[c].at[pl.ds(b, tail)])


def _dec_stage(p1_pair, p2_pair, r1, r2):
    l = r1.shape[0]
    assert l % 8 == 0
    tail_n = max(l % _EC, 8)
    mesh = plsc.VectorSubcoreMesh(core_axis_name="c", subcore_axis_name="s")
    fn = pl.kernel(
        _dec_body,
        out_type=jax.ShapeDtypeStruct((2, l, HH), jnp.float32),
        mesh=mesh,
        scratch_types=[
            pltpu.VMEM((2, _EC), jnp.int32),
            pltpu.VMEM((2, _EC), jnp.int32),
            pltpu.VMEM((2, _EC, HH), jnp.float32),
            pltpu.VMEM((2, _EC, HH), jnp.float32),
            pltpu.VMEM((tail_n,), jnp.int32),
            pltpu.VMEM((tail_n,), jnp.int32),
            pltpu.VMEM((tail_n, HH), jnp.float32),
            pltpu.VMEM((tail_n, HH), jnp.float32),
            pltpu.SemaphoreType.DMA,
            pltpu.SemaphoreType.DMA,
            pltpu.SemaphoreType.DMA,
            pltpu.SemaphoreType.DMA,
            pltpu.SemaphoreType.DMA,
            pltpu.SemaphoreType.DMA,
        ],
    )
    return fn(p1_pair, p2_pair, r1, r2)


def kernel(x_cellline, x_drug, edge_index_cd, edge_index_dc, edge_weight_cd,
           edge_weight_dc, edge_label_index, W1_cd, W2_cd, W3_cd, W1_dc,
           W2_dc, W3_dc, b1_cd, b3_cd, b1_dc, b3_dc, Wd1, bd1, Wd2, bd2):
    zeros = jnp.zeros((H,), jnp.float32)
    Wc = jnp.concatenate([W1_cd, W2_dc, W3_dc], axis=1)
    bc = jnp.concatenate([b1_cd, zeros, b3_dc])
    Wd = jnp.concatenate([W1_dc, W2_cd, W3_cd], axis=1)
    bd = jnp.concatenate([b1_dc, zeros, b3_cd])
    pc, a_cd_pair = _matmul_bias(x_cellline, Wc, bc)
    pd, a_dc_pair = _matmul_bias(x_drug, Wd, bd)

    s_cd_pair, degw_cd, s_dc_pair, degw_dc = _edge_stage(
        a_cd_pair, edge_index_cd[0], edge_index_cd[1], edge_weight_cd,
        a_dc_pair, edge_index_dc[0], edge_index_dc[1], edge_weight_dc)

    p1_pair = _p_matmul(s_dc_pair, pc, 2, 1, degw_dc, Wd1[:H], bd1)
    p2_pair = _p_matmul(s_cd_pair, pd, 2, 1, degw_cd, Wd1[H:],
                        jnp.zeros((H,), jnp.float32))

    g_pair = _dec_stage(p1_pair, p2_pair,
                        edge_label_index[0], edge_label_index[1])
    return _final_stage(g_pair, Wd2.reshape(-1), bd2)


# speedup vs baseline: 1.0417x; 1.0417x over previous
import jax
import jax.numpy as jnp
from jax import lax
from jax.experimental import pallas as pl
from jax.experimental.pallas import tpu as pltpu
from jax.experimental.pallas import tpu_sc as plsc

H = 256
HH = 128


def _mm_kernel(x_ref, w_ref, b_ref, o_ref, o2_ref):
    res = (
        jnp.dot(x_ref[...], w_ref[...], preferred_element_type=jnp.float32)
        + b_ref[...]
    )
    o_ref[...] = res
    o2_ref[0] = res[:, :HH]
    o2_ref[1] = res[:, HH:2 * HH]


def _matmul_bias(x, w, b, block_m=400):
    m, k = x.shape
    n = w.shape[1]
    assert m % block_m == 0
    return pl.pallas_call(
        _mm_kernel,
        grid=(m // block_m,),
        in_specs=[
            pl.BlockSpec((block_m, k), lambda i: (i, 0)),
            pl.BlockSpec((k, n), lambda i: (0, 0)),
            pl.BlockSpec((1, n), lambda i: (0, 0)),
        ],
        out_specs=[
            pl.BlockSpec((block_m, n), lambda i: (i, 0)),
            pl.BlockSpec((2, block_m, HH), lambda i: (0, i, 0)),
        ],
        out_shape=[
            jax.ShapeDtypeStruct((m, n), jnp.float32),
            jax.ShapeDtypeStruct((2, m, HH), jnp.float32),
        ],
    )(x, w, b.reshape(1, n))


def _pz_kernel(s_ref, t_ref, b_ref, dg_ref, w_ref, bias_ref, o_ref):
    z = (jnp.concatenate([s_ref[0], s_ref[1]], axis=-1)
         + t_ref[...] - b_ref[...] * dg_ref[...])
    o_ref[0] = (
        jnp.dot(z, w_ref[...], preferred_element_type=jnp.float32)
        + bias_ref[0]
    )


def _p_matmul(s_pair, pcd, t_col, b_col, degw, w, bias, block_m=400):
    n = s_pair.shape[1]
    assert n % block_m == 0
    return pl.pallas_call(
        _pz_kernel,
        grid=(n // block_m, 2),
        in_specs=[
            pl.BlockSpec((2, block_m, HH), lambda i, j: (0, i, 0)),
            pl.BlockSpec((block_m, H), lambda i, j, c=t_col: (i, c)),
            pl.BlockSpec((block_m, H), lambda i, j, c=b_col: (i, c)),
            pl.BlockSpec((block_m, 1), lambda i, j: (i, 0)),
            pl.BlockSpec((H, HH), lambda i, j: (0, j)),
            pl.BlockSpec((1, 1, HH), lambda i, j: (j, 0, 0)),
        ],
        out_specs=pl.BlockSpec((1, block_m, HH), lambda i, j: (j, i, 0)),
        out_shape=jax.ShapeDtypeStruct((2, n, HH), jnp.float32),
    )(s_pair, pcd, pcd, degw.reshape(n, 1), w, bias.reshape(2, 1, HH))


def _fin_kernel(g_ref, w2_ref, b2_ref, o_ref):
    q = jax.nn.sigmoid(jnp.concatenate([g_ref[0], g_ref[1]], axis=-1))
    o_ref[...] = jax.nn.sigmoid(
        jnp.sum(q * w2_ref[...], axis=1, keepdims=True) + b2_ref[...])


def _final_stage(g_pair, w2, b2, block_m=2000):
    m = g_pair.shape[1]
    assert m % block_m == 0
    out = pl.pallas_call(
        _fin_kernel,
        grid=(m // block_m,),
        in_specs=[
            pl.BlockSpec((2, block_m, HH), lambda i: (0, i, 0)),
            pl.BlockSpec((1, H), lambda i: (0, 0)),
            pl.BlockSpec((1, 1), lambda i: (0, 0)),
        ],
        out_specs=pl.BlockSpec((block_m, 1), lambda i: (i, 0)),
        out_shape=jax.ShapeDtypeStruct((m, 1), jnp.float32),
    )(g_pair, w2.reshape(1, H), b2.reshape(1, 1))
    return out.reshape(-1)


_EC = 128


def _edge_phase(a_hbm, src_hbm, dst_hbm, w_hbm, s_out, degw_out,
                src_v, dst_v, w_v, rows_v, src_t, dst_t, w_t, rows_t,
                zdeg_v, acc, degw_acc, dvs_v, wvs_v,
                sem_i0, sem_i1, sem_g0, sem_g1, sem_s0, sem_s1,
                sem_d0, sem_d1):
    c = lax.axis_index("c")
    s = lax.axis_index("s")
    n = a_hbm.shape[1]
    e = src_hbm.shape[0]
    per_sub = e // 16
    nfull = per_sub // _EC
    tail = per_sub - nfull * _EC
    rows_a = (n // 16) // 8 * 8
    rows_rem = n - 16 * rows_a
    nzc = rows_a // _EC
    zrem = rows_a - nzc * _EC

    zero16 = jnp.zeros((16,), jnp.float32)

    def _zrow2(i, _):
        rows_v[0, i // 8, pl.ds((i % 8) * 16, 16)] = zero16
        rows_v[1, i // 8, pl.ds((i % 8) * 16, 16)] = zero16
        return 0
    lax.fori_loop(0, _EC * 8, _zrow2, 0)

    def _zdeg(i, _):
        zdeg_v[pl.ds(i * 16, 16)] = zero16
        return 0
    lax.fori_loop(0, zdeg_v.shape[0] // 16, _zdeg, 0)

    for q in range(nzc):
        pltpu.sync_copy(rows_v.at[0],
                        acc.at[pl.ds(s * rows_a + q * _EC, _EC)])
    pltpu.sync_copy(rows_v.at[0].at[pl.ds(0, zrem)],
                    acc.at[pl.ds(s * rows_a + nzc * _EC, zrem)])

    @pl.when(s == 15)
    def _():
        pltpu.sync_copy(rows_v.at[0].at[pl.ds(0, rows_rem)],
                        acc.at[pl.ds(16 * rows_a, rows_rem)])

    @pl.when(jnp.logical_and(c == 0, s < 10))
    def _():
        pltpu.sync_copy(zdeg_v.at[pl.ds(0, 1000)],
                        degw_acc.at[pl.ds(s * 1000, 1000)])

    plsc.subcore_barrier()

    def _scale(rows, w_ref, k):
        def body(r16, _):
            wrow = w_ref[pl.ds(r16 * 16, 16)]
            for t in range(16):
                r = r16 * 16 + t
                wv = lax.gather(
                    wrow, jnp.full((16, 1), t, jnp.int32),
                    lax.GatherDimensionNumbers(
                        offset_dims=(), collapsed_slice_dims=(0,),
                        start_index_map=(0,)),
                    (1,), mode=lax.GatherScatterMode.PROMISE_IN_BOUNDS)
                for j in range(8):
                    sl = pl.ds(j * 16, 16)
                    rows[r, sl] = rows[r, sl] * wv
            return 0
        lax.fori_loop(0, k // 16, body, 0)

    sv = (src_v.at[0], src_v.at[1])
    dv = (dst_v.at[0], dst_v.at[1])
    wv = (w_v.at[0], w_v.at[1])
    rv = (rows_v.at[0], rows_v.at[1])
    dvs = (dvs_v.at[0], dvs_v.at[1])
    wvs = (wvs_v.at[0], wvs_v.at[1])
    semi = (sem_i0, sem_i1)
    semg = (sem_g0, sem_g1)
    sems = (sem_s0, sem_s1)
    semd = (sem_d0, sem_d1)

    def base_of(g):
        return s * per_sub + jnp.minimum(g, nfull - 1) * _EC

    def fetch_idx(g, p):
        b = base_of(g)
        pltpu.async_copy(src_hbm.at[pl.ds(b, _EC)], sv[p], semi[p])
        pltpu.async_copy(dst_hbm.at[pl.ds(b, _EC)], dv[p], semi[p])
        pltpu.async_copy(w_hbm.at[pl.ds(b, _EC)], wv[p], semi[p])

    def wait_idx(p):
        b = base_of(0)
        pltpu.make_async_copy(src_hbm.at[pl.ds(b, _EC)], sv[p], semi[p]).wait()
        pltpu.make_async_copy(dst_hbm.at[pl.ds(b, _EC)], dv[p], semi[p]).wait()
        pltpu.make_async_copy(w_hbm.at[pl.ds(b, _EC)], wv[p], semi[p]).wait()

    def gather_rows(p):
        pltpu.async_copy(a_hbm.at[c].at[sv[p]], rv[p], semg[p])

    def wait_rows(p):
        pltpu.make_async_copy(a_hbm.at[c].at[sv[p]], rv[p], semg[p]).wait()

    def snap_idx(p):
        for j in range(8):
            sl = pl.ds(j * 16, 16)
            dvs[p][sl] = dv[p][sl]

        @pl.when(c == 0)
        def _():
            for j in range(8):
                sl = pl.ds(j * 16, 16)
                wvs[p][sl] = wv[p][sl]

    def scatter(p):
        pltpu.async_copy(rv[p], acc.at[dvs[p]], sems[p], add=True)

        @pl.when(c == 0)
        def _():
            pltpu.async_copy(wvs[p], degw_acc.at[dvs[p]], semd[p], add=True)

    def wait_scatter(p):
        pltpu.make_async_copy(rv[p], acc.at[dvs[p]], sems[p]).wait()

        @pl.when(c == 0)
        def _():
            pltpu.make_async_copy(wvs[p], degw_acc.at[dvs[p]],
                                  semd[p]).wait()

    def half_step(i, p):
        wait_rows(p)
        _scale(rv[p], wv[p], _EC)
        snap_idx(p)
        wait_idx(1 - p)

        @pl.when(i >= 1)
        def _():
            wait_scatter(1 - p)
        gather_rows(1 - p)
        scatter(p)
        fetch_idx(i + 2, p)

    assert nfull % 2 == 0
    fetch_idx(0, 0)
    wait_idx(0)
    gather_rows(0)
    fetch_idx(1, 1)

    def _main(u, _):
        half_step(2 * u, 0)
        half_step(2 * u + 1, 1)
        return 0
    lax.fori_loop(0, nfull // 2, _main, 0)
    wait_rows(0)
    wait_idx(1)
    wait_scatter(1)

    if tail:
        b = s * per_sub + nfull * _EC
        pltpu.sync_copy(src_hbm.at[pl.ds(b, tail)], src_t)
        pltpu.sync_copy(dst_hbm.at[pl.ds(b, tail)], dst_t)
        pltpu.sync_copy(w_hbm.at[pl.ds(b, tail)], w_t)
        pltpu.async_copy(a_hbm.at[c].at[src_t], rows_t, sem_g0).wait()
        _scale(rows_t, w_t, tail)
        pltpu.sync_copy(rows_t, acc.at[dst_t], add=True)

        @pl.when(c == 0)
        def _():
            pltpu.sync_copy(w_t, degw_acc.at[dst_t], add=True)

    plsc.subcore_barrier()

    for q in range(nzc):
        lo = s * rows_a + q * _EC
        pltpu.sync_copy(acc.at[pl.ds(lo, _EC)], rows_v.at[0])
        pltpu.sync_copy(rows_v.at[0], s_out.at[c].at[pl.ds(lo, _EC)])
    lo2 = s * rows_a + nzc * _EC
    pltpu.sync_copy(acc.at[pl.ds(lo2, zrem)], rows_v.at[0].at[pl.ds(0, zrem)])
    pltpu.sync_copy(rows_v.at[0].at[pl.ds(0, zrem)],
                    s_out.at[c].at[pl.ds(lo2, zrem)])

    @pl.when(s == 15)
    def _():
        pltpu.sync_copy(acc.at[pl.ds(16 * rows_a, rows_rem)],
                        rows_v.at[1].at[pl.ds(0, rows_rem)])
        pltpu.sync_copy(rows_v.at[1].at[pl.ds(0, rows_rem)],
                        s_out.at[c].at[pl.ds(16 * rows_a, rows_rem)])

    @pl.when(jnp.logical_and(c == 0, s < 10))
    def _():
        pltpu.sync_copy(degw_acc.at[pl.ds(s * 1000, 1000)],
                        zdeg_v.at[pl.ds(0, 1000)])
        pltpu.sync_copy(zdeg_v.at[pl.ds(0, 1000)],
                        degw_out.at[pl.ds(s * 1000, 1000)])


def _edge_stage(a_pair, src, dst, w):
    n = a_pair.shape[1]
    e = src.shape[0]
    assert e % 16 == 0 and (e // 16) % 8 == 0
    assert n % 16 == 0 and n % 1000 == 0
    tail_n = max(e // 16 % _EC, 8)
    mesh = plsc.VectorSubcoreMesh(core_axis_name="c", subcore_axis_name="s")
    fn = pl.kernel(
        _edge_phase,
        out_type=[
            jax.ShapeDtypeStruct((2, n, HH), jnp.float32),
            jax.ShapeDtypeStruct((n,), jnp.float32),
        ],
        mesh=mesh,
        scratch_types=[
            pltpu.VMEM((2, _EC), jnp.int32),
            pltpu.VMEM((2, _EC), jnp.int32),
            pltpu.VMEM((2, _EC), jnp.float32),
            pltpu.VMEM((2, _EC, HH), jnp.float32),
            pltpu.VMEM((tail_n,), jnp.int32),
            pltpu.VMEM((tail_n,), jnp.int32),
            pltpu.VMEM((tail_n,), jnp.float32),
            pltpu.VMEM((tail_n, HH), jnp.float32),
            pltpu.VMEM((1040,), jnp.float32),
            pltpu.VMEM_SHARED((n, HH), jnp.float32),
            pltpu.VMEM_SHARED((n,), jnp.float32),
            pltpu.VMEM((2, _EC), jnp.int32),
            pltpu.VMEM((2, _EC), jnp.float32),
            pltpu.SemaphoreType.DMA,
            pltpu.SemaphoreType.DMA,
            pltpu.SemaphoreType.DMA,
            pltpu.SemaphoreType.DMA,
            pltpu.SemaphoreType.DMA,
            pltpu.SemaphoreType.DMA,
            pltpu.SemaphoreType.DMA,
            pltpu.SemaphoreType.DMA,
        ],
    )
    return fn(a_pair, src, dst, w)


def _dec_body(p1_hbm, p2_hbm, r1_hbm, r2_hbm, g_out,
              i1_v, i2_v, g1_v, g2_v, i1_t, i2_t, g1_t, g2_t,
              sem_i0, sem_i1, sem_g0, sem_g1, sem_o0, sem_o1):
    c = lax.axis_index("c")
    s = lax.axis_index("s")
    l = r1_hbm.shape[0]
    nfull = l // _EC
    tail = l - nfull * _EC
    n_t = (nfull - s + 15) // 16

    i1 = (i1_v.at[0], i1_v.at[1])
    i2 = (i2_v.at[0], i2_v.at[1])
    g1 = (g1_v.at[0], g1_v.at[1])
    g2 = (g2_v.at[0], g2_v.at[1])
    semi = (sem_i0, sem_i1)
    semg = (sem_g0, sem_g1)
    semo = (sem_o0, sem_o1)

    def base_of(t):
        return (s + 16 * jnp.minimum(t, n_t - 1)) * _EC

    def _add(ga, gb, k):
        def body(r, _):
            for j in range(8):
                sl = pl.ds(j * 16, 16)
                ga[r, sl] = ga[r, sl] + gb[r, sl]
            return 0
        lax.fori_loop(0, k, body, 0)

    def fetch_idx(t, p):
        b = base_of(t)
        pltpu.async_copy(r1_hbm.at[pl.ds(b, _EC)], i1[p], semi[p])
        pltpu.async_copy(r2_hbm.at[pl.ds(b, _EC)], i2[p], semi[p])

    def wait_idx(p):
        pltpu.make_async_copy(r1_hbm.at[pl.ds(0, _EC)], i1[p], semi[p]).wait()
        pltpu.make_async_copy(r2_hbm.at[pl.ds(0, _EC)], i2[p], semi[p]).wait()

    def gathers(p):
        pltpu.async_copy(p1_hbm.at[c].at[i1[p]], g1[p], semg[p])
        pltpu.async_copy(p2_hbm.at[c].at[i2[p]], g2[p], semg[p])

    def wait_gathers(p):
        pltpu.make_async_copy(p1_hbm.at[c].at[i1[p]], g1[p], semg[p]).wait()
        pltpu.make_async_copy(p2_hbm.at[c].at[i2[p]], g2[p], semg[p]).wait()

    def wait_store(p):
        pltpu.make_async_copy(g_out.at[c].at[pl.ds(0, _EC)],
                              g1[p], semo[p]).wait()

    def step(t, p):
        wait_gathers(p)
        _add(g1[p], g2[p], _EC)
        wait_idx(1 - p)

        @pl.when(t >= 1)
        def _():
            wait_store(1 - p)
        gathers(1 - p)
        pltpu.async_copy(g1[p], g_out.at[c].at[pl.ds(base_of(t), _EC)],
                         semo[p])
        fetch_idx(t + 2, p)

    fetch_idx(0, 0)
    wait_idx(0)
    gathers(0)
    fetch_idx(1, 1)

    nslots = (((nfull + 15) // 16) + 1) // 2 * 2

    def _main(u, _):
        step(2 * u, 0)
        step(2 * u + 1, 1)
        return 0
    lax.fori_loop(0, nslots // 2, _main, 0)
    wait_gathers(0)
    wait_idx(1)
    wait_store(1)

    if tail:
        @pl.when(s == 15)
        def _():
            b = nfull * _EC
            pltpu.sync_copy(r1_hbm.at[pl.ds(b, tail)], i1_t)
            pltpu.sync_copy(r2_hbm.at[pl.ds(b, tail)], i2_t)
            pltpu.async_copy(p1_hbm.at[c].at[i1_t], g1_t, sem_g0).wait()
            pltpu.async_copy(p2_hbm.at[c].at[i2_t], g2_t, sem_g0).wait()
            _add(g1_t, g2_t, tail)
            pltpu.sync_copy(g1_t, g_out.at[c].at[pl.ds(b, tail)])


def _dec_stage(p1_pair, p2_pair, r1, r2):
    l = r1.shape[0]
    assert l % 8 == 0
    tail_n = max(l % _EC, 8)
    mesh = plsc.VectorSubcoreMesh(core_axis_name="c", subcore_axis_name="s")
    fn = pl.kernel(
        _dec_body,
        out_type=jax.ShapeDtypeStruct((2, l, HH), jnp.float32),
        mesh=mesh,
        scratch_types=[
            pltpu.VMEM((2, _EC), jnp.int32),
            pltpu.VMEM((2, _EC), jnp.int32),
            pltpu.VMEM((2, _EC, HH), jnp.float32),
            pltpu.VMEM((2, _EC, HH), jnp.float32),
            pltpu.VMEM((tail_n,), jnp.int32),
            pltpu.VMEM((tail_n,), jnp.int32),
            pltpu.VMEM((tail_n, HH), jnp.float32),
            pltpu.VMEM((tail_n, HH), jnp.float32),
            pltpu.SemaphoreType.DMA,
            pltpu.SemaphoreType.DMA,
            pltpu.SemaphoreType.DMA,
            pltpu.SemaphoreType.DMA,
            pltpu.SemaphoreType.DMA,
            pltpu.SemaphoreType.DMA,
        ],
    )
    return fn(p1_pair, p2_pair, r1, r2)


def kernel(x_cellline, x_drug, edge_index_cd, edge_index_dc, edge_weight_cd,
           edge_weight_dc, edge_label_index, W1_cd, W2_cd, W3_cd, W1_dc,
           W2_dc, W3_dc, b1_cd, b3_cd, b1_dc, b3_dc, Wd1, bd1, Wd2, bd2):
    zeros = jnp.zeros((H,), jnp.float32)
    Wc = jnp.concatenate([W1_cd, W2_dc, W3_dc], axis=1)
    bc = jnp.concatenate([b1_cd, zeros, b3_dc])
    Wd = jnp.concatenate([W1_dc, W2_cd, W3_cd], axis=1)
    bd = jnp.concatenate([b1_dc, zeros, b3_cd])
    pc, a_cd_pair = _matmul_bias(x_cellline, Wc, bc)
    pd, a_dc_pair = _matmul_bias(x_drug, Wd, bd)

    s_cd_pair, degw_cd = _edge_stage(
        a_cd_pair, edge_index_cd[0], edge_index_cd[1], edge_weight_cd)
    s_dc_pair, degw_dc = _edge_stage(
        a_dc_pair, edge_index_dc[0], edge_index_dc[1], edge_weight_dc)

    p1_pair = _p_matmul(s_dc_pair, pc, 2, 1, degw_dc, Wd1[:H], bd1)
    p2_pair = _p_matmul(s_cd_pair, pd, 2, 1, degw_cd, Wd1[H:],
                        jnp.zeros((H,), jnp.float32))

    g_pair = _dec_stage(p1_pair, p2_pair,
                        edge_label_index[0], edge_label_index[1])
    return _final_stage(g_pair, Wd2.reshape(-1), bd2)
